# Initial kernel scaffold; baseline (speedup 1.0000x reference)
import jax
import jax.numpy as jnp
from jax.experimental import pallas as pl


def kernel(gt_inds, anchor_gt_inds, gt_bboxes, proposal_list):
    raise NotImplementedError("write your pallas kernel here")


# speedup vs baseline: 99.3980x; 99.3980x over previous
import jax
import jax.numpy as jnp
from jax.experimental import pallas as pl
from jax.experimental.pallas import tpu as pltpu

NMS_THR = 0.5
MIN_HEIGHT = 50.0
EPS = 1e-6
N = 5000
G = 128
NP = 5120
TB = 512
W16 = NP // 16
WB = TB // 16


def _pack_body(x1c, y1c, x2c, y2c, x1r, y1r, x2r, y2r, m_ref):
    ib = pl.program_id(0)
    x1i = x1c[...]; y1i = y1c[...]; x2i = x2c[...]; y2i = y2c[...]
    a_i = (x2i - x1i) * (y2i - y1i)
    row_iota = jax.lax.broadcasted_iota(jnp.int32, (TB, TB), 0)
    col_iota = jax.lax.broadcasted_iota(jnp.int32, (TB, TB), 1)
    ci = jax.lax.broadcasted_iota(jnp.int32, (TB, WB), 0)
    wi = jax.lax.broadcasted_iota(jnp.int32, (TB, WB), 1)
    pmat = jnp.where((ci // 16) == wi,
                     jnp.left_shift(jnp.int32(1), ci % 16), 0).astype(jnp.float32)

    for jb in range(NP // TB):
        sl = slice(jb * WB, (jb + 1) * WB)

        @pl.when(jb >= ib)
        def _(jb=jb, sl=sl):
            cs = slice(jb * TB, (jb + 1) * TB)
            x1j = x1r[:, cs]; y1j = y1r[:, cs]
            x2j = x2r[:, cs]; y2j = y2r[:, cs]
            a_j = (x2j - x1j) * (y2j - y1j)
            ltx = jnp.maximum(x1i, x1j)
            lty = jnp.maximum(y1i, y1j)
            rbx = jnp.minimum(x2i, x2j)
            rby = jnp.minimum(y2i, y2j)
            wx = jnp.clip(rbx - ltx, 0.0, None)
            wy = jnp.clip(rby - lty, 0.0, None)
            inter = wx * wy
            union = a_i + a_j - inter
            iou = inter / jnp.maximum(union, EPS)
            row_i = ib * TB + row_iota
            col_j = jb * TB + col_iota
            over = (iou > NMS_THR) & (col_j > row_i) & (col_j < N)
            bits = jax.lax.dot_general(over.astype(jnp.float32), pmat,
                                       (((1,), (0,)), ((), ())),
                                       preferred_element_type=jnp.float32)
            m_ref[:, sl] = bits.astype(jnp.int32)

        @pl.when(jb < ib)
        def _(sl=sl):
            m_ref[:, sl] = jnp.zeros((TB, WB), jnp.int32)


def _aux_body(x1r, y1r, x2r, y2r, scr, gir, orr, gx1, gy1, gx2, gy2,
              pl_out, pb_out, kp_out):
    gidx = jax.lax.broadcasted_iota(jnp.int32, (G, 1), 0)
    gi = gir[...]
    eq = gi == gidx
    ggx1 = jnp.sum(jnp.where(eq, gx1[...], 0.0), axis=0, keepdims=True)
    ggy1 = jnp.sum(jnp.where(eq, gy1[...], 0.0), axis=0, keepdims=True)
    ggx2 = jnp.sum(jnp.where(eq, gx2[...], 0.0), axis=0, keepdims=True)
    ggy2 = jnp.sum(jnp.where(eq, gy2[...], 0.0), axis=0, keepdims=True)
    bx1 = x1r[...]; by1 = y1r[...]; bx2 = x2r[...]; by2 = y2r[...]
    area_g = (ggx2 - ggx1) * (ggy2 - ggy1)
    area_b = (bx2 - bx1) * (by2 - by1)
    ltx = jnp.maximum(ggx1, bx1)
    lty = jnp.maximum(ggy1, by1)
    rbx = jnp.minimum(ggx2, bx2)
    rby = jnp.minimum(ggy2, by2)
    wx = jnp.clip(rbx - ltx, 0.0, None)
    wy = jnp.clip(rby - lty, 0.0, None)
    inter = wx * wy
    msi = inter / jnp.maximum(area_g + area_b - inter, EPS)
    msi = jnp.clip(msi, EPS, None)
    pl_out[...] = -jnp.log(jnp.clip(1.0 - NMS_THR + msi, None, 1.0)) * scr[...]
    minf = jnp.float32(-jnp.inf)
    msc = jnp.where(eq, scr[...], minf)
    best = jnp.max(msc, axis=1, keepdims=True)
    cand2 = eq & (msc == best)
    oid = orr[...]
    pi = jnp.min(jnp.where(cand2, oid, N), axis=1, keepdims=True)
    sel = cand2 & (oid == pi)
    sx1 = jnp.sum(jnp.where(sel, bx1, 0.0), axis=1, keepdims=True)
    sy1 = jnp.sum(jnp.where(sel, by1, 0.0), axis=1, keepdims=True)
    sx2 = jnp.sum(jnp.where(sel, bx2, 0.0), axis=1, keepdims=True)
    sy2 = jnp.sum(jnp.where(sel, by2, 0.0), axis=1, keepdims=True)
    g_area = (gx2[...] - gx1[...]) * (gy2[...] - gy1[...])
    s_area = (sx2 - sx1) * (sy2 - sy1)
    ltx2 = jnp.maximum(gx1[...], sx1)
    lty2 = jnp.maximum(gy1[...], sy1)
    rbx2 = jnp.minimum(gx2[...], sx2)
    rby2 = jnp.minimum(gy2[...], sy2)
    wx2 = jnp.clip(rbx2 - ltx2, 0.0, None)
    wy2 = jnp.clip(rby2 - lty2, 0.0, None)
    inter2 = wx2 * wy2
    gsel = inter2 / jnp.maximum(g_area + s_area - inter2, EPS)
    has = jnp.any(eq, axis=1, keepdims=True)
    height = gy2[...] - gy1[...]
    kp_out[...] = (has & (height >= MIN_HEIGHT)).astype(jnp.int32)
    pb_out[...] = jnp.where(has, 1.0 - gsel, 0.0)


def _scan_body(gi_sref, pl_sref, m_ref, seen_out, scal_out, acc_ref, accp_ref):
    def init_g(g, c):
        seen_out[0, g] = 0
        return c
    jax.lax.fori_loop(0, G, init_g, 0)
    acc_ref[...] = jnp.zeros((1, W16), jnp.int32)
    accp_ref[...] = jnp.zeros((1, W16), jnp.int32)
    lanes = jax.lax.broadcasted_iota(jnp.int32, (1, W16), 1)

    def body(p, carry):
        pcnt, psum, l_cnt, l_pl, l_p = carry
        acc = acc_ref[...]
        w = p // 16
        b = p - w * 16
        word = jnp.sum(acc * (lanes == w).astype(jnp.int32))
        kept = ((word >> b) & 1) == 0
        gv = gi_sref[0, p]
        counted = seen_out[0, gv]
        keptc = kept & (counted > 0)
        row = m_ref[pl.ds(p, 1), :]
        kmask = -kept.astype(jnp.int32)
        accp = accp_ref[...]
        accp_ref[...] = accp ^ ((accp ^ acc) & kmask)
        acc_ref[...] = acc | (row & kmask)
        seen_out[0, gv] = jnp.where(kept, 1, counted)
        plv = pl_sref[0, p]
        pcnt = pcnt + jnp.where(keptc, 1, 0).astype(jnp.int32)
        psum = psum + jnp.where(keptc, plv, 0.0)
        l_cnt = jnp.where(kept, keptc.astype(jnp.int32), l_cnt)
        l_pl = jnp.where(kept, plv, l_pl)
        l_p = jnp.where(kept, p, l_p)
        return pcnt, psum, l_cnt, l_pl, l_p

    pcnt, psum, l_cnt, l_pl, l_p = jax.lax.fori_loop(
        0, N, body,
        (jnp.int32(0), jnp.float32(0.0),
         jnp.int32(0), jnp.float32(0.0), jnp.int32(0)))
    accp = accp_ref[...]
    base = lanes * 16
    s1 = jnp.clip((l_p + 1) - base, 0, 16)
    mask_lo = jnp.left_shift(jnp.int32(0xFFFF), s1) & 0xFFFF
    s2 = jnp.clip(N - base, 0, 16)
    mask_hi = jnp.left_shift(jnp.int32(1), s2) - 1
    zeros_alive = jnp.bitwise_not(accp) & (mask_lo & mask_hi)
    l_rem = jnp.max(zeros_alive) > 0
    scal_out[0, 0] = pcnt.astype(jnp.float32)
    scal_out[0, 1] = psum
    scal_out[0, 2] = l_cnt.astype(jnp.float32)
    scal_out[0, 3] = l_rem.astype(jnp.float32)
    scal_out[0, 4] = l_pl


def _pack_call(x1c, y1c, x2c, y2c, x1r, y1r, x2r, y2r):
    col = pl.BlockSpec((TB, 1), lambda i: (i, 0))
    row = pl.BlockSpec((1, NP), lambda i: (0, 0))
    return pl.pallas_call(
        _pack_body,
        grid=(NP // TB,),
        in_specs=[col, col, col, col, row, row, row, row],
        out_specs=pl.BlockSpec((TB, W16), lambda i: (i, 0)),
        out_shape=jax.ShapeDtypeStruct((NP, W16), jnp.int32),
    )(x1c, y1c, x2c, y2c, x1r, y1r, x2r, y2r)


def _aux_call(x1r, y1r, x2r, y2r, scr, gir, orr, gx1, gy1, gx2, gy2):
    return pl.pallas_call(
        _aux_body,
        in_specs=[pl.BlockSpec()] * 11,
        out_specs=[pl.BlockSpec(), pl.BlockSpec(), pl.BlockSpec()],
        out_shape=[
            jax.ShapeDtypeStruct((1, NP), jnp.float32),
            jax.ShapeDtypeStruct((G, 1), jnp.float32),
            jax.ShapeDtypeStruct((G, 1), jnp.int32),
        ],
    )(x1r, y1r, x2r, y2r, scr, gir, orr, gx1, gy1, gx2, gy2)


def _scan_call(gir, plv, m):
    smem = pl.BlockSpec(memory_space=pltpu.SMEM)
    return pl.pallas_call(
        _scan_body,
        in_specs=[smem, smem, pl.BlockSpec()],
        out_specs=[smem, smem],
        out_shape=[
            jax.ShapeDtypeStruct((1, G), jnp.int32),
            jax.ShapeDtypeStruct((1, 8), jnp.float32),
        ],
        scratch_shapes=[
            pltpu.VMEM((1, W16), jnp.int32),
            pltpu.VMEM((1, W16), jnp.int32),
        ],
    )(gir, plv, m)


def kernel(gt_inds, anchor_gt_inds, gt_bboxes, proposal_list):
    del gt_inds
    gi0 = anchor_gt_inds[0].astype(jnp.int32)
    props = proposal_list[0]
    gtb = gt_bboxes[0]
    scores = props[:, 4]
    order = jnp.argsort(scores)[::-1].astype(jnp.int32)
    bs = props[order]
    gis = gi0[order]
    pad = NP - N
    x1 = jnp.pad(bs[:, 0], (0, pad))
    y1 = jnp.pad(bs[:, 1], (0, pad))
    x2 = jnp.pad(bs[:, 2], (0, pad))
    y2 = jnp.pad(bs[:, 3], (0, pad))
    sc = jnp.pad(bs[:, 4], (0, pad))
    gip = jnp.pad(gis, (0, pad), constant_values=-1)
    orp = jnp.pad(order, (0, pad), constant_values=N)
    x1c, y1c, x2c, y2c = (a[:, None] for a in (x1, y1, x2, y2))
    x1r, y1r, x2r, y2r, scr = (a[None, :] for a in (x1, y1, x2, y2, sc))
    gir = gip[None, :]
    orr = orp[None, :]
    gx1, gy1, gx2, gy2 = (gtb[:, k][:, None] for k in range(4))

    m = _pack_call(x1c, y1c, x2c, y2c, x1r, y1r, x2r, y2r)
    plv, pb, kp = _aux_call(x1r, y1r, x2r, y2r, scr, gir, orr, gx1, gy1, gx2, gy2)
    seen, scal = _scan_call(gir, plv, m)

    pull_cnt = scal[0, 0]
    psum = scal[0, 1]
    l_cnt = scal[0, 2]
    l_rem = scal[0, 3]
    l_pl = scal[0, 4]
    total_pull = psum - jnp.where((l_cnt > 0) & (l_rem == 0), l_pl, 0.0)
    pull_loss = total_pull / (pull_cnt + EPS)
    keepg = (kp[:, 0] > 0) & (seen[0, :] == 0)
    total_push = jnp.sum(jnp.where(keepg, pb[:, 0], 0.0))
    push_cnt = jnp.sum(keepg)
    push_loss = total_push / (push_cnt + EPS)
    return jnp.stack([push_loss, pull_loss])


# speedup vs baseline: 247.2769x; 2.4877x over previous
import jax
import jax.numpy as jnp
from jax.experimental import pallas as pl
from jax.experimental.pallas import tpu as pltpu

NMS_THR = 0.5
MIN_HEIGHT = 50.0
EPS = 1e-6
N = 5000
G = 128
NP = 5120
TB = 512
W16 = NP // 16
WB = TB // 16


def _pack_body(x1c, y1c, x2c, y2c, x1r, y1r, x2r, y2r, m_ref):
    ib = pl.program_id(0)
    x1i = x1c[...]; y1i = y1c[...]; x2i = x2c[...]; y2i = y2c[...]
    a_i = (x2i - x1i) * (y2i - y1i)
    row_iota = jax.lax.broadcasted_iota(jnp.int32, (TB, TB), 0)
    col_iota = jax.lax.broadcasted_iota(jnp.int32, (TB, TB), 1)
    ci = jax.lax.broadcasted_iota(jnp.int32, (TB, WB), 0)
    wi = jax.lax.broadcasted_iota(jnp.int32, (TB, WB), 1)
    pmat = jnp.where((ci // 16) == wi,
                     jnp.left_shift(jnp.int32(1), ci % 16), 0).astype(jnp.float32)

    for jb in range(NP // TB):
        sl = slice(jb * WB, (jb + 1) * WB)

        @pl.when(jb >= ib)
        def _(jb=jb, sl=sl):
            cs = slice(jb * TB, (jb + 1) * TB)
            x1j = x1r[:, cs]; y1j = y1r[:, cs]
            x2j = x2r[:, cs]; y2j = y2r[:, cs]
            a_j = (x2j - x1j) * (y2j - y1j)
            ltx = jnp.maximum(x1i, x1j)
            lty = jnp.maximum(y1i, y1j)
            rbx = jnp.minimum(x2i, x2j)
            rby = jnp.minimum(y2i, y2j)
            wx = jnp.clip(rbx - ltx, 0.0, None)
            wy = jnp.clip(rby - lty, 0.0, None)
            inter = wx * wy
            union = a_i + a_j - inter
            iou = inter / jnp.maximum(union, EPS)
            row_i = ib * TB + row_iota
            col_j = jb * TB + col_iota
            over = (iou > NMS_THR) & (col_j > row_i) & (col_j < N)
            bits = jax.lax.dot_general(over.astype(jnp.float32), pmat,
                                       (((1,), (0,)), ((), ())),
                                       preferred_element_type=jnp.float32)
            m_ref[:, sl] = bits.astype(jnp.int32)

        @pl.when(jb < ib)
        def _(sl=sl):
            m_ref[:, sl] = jnp.zeros((TB, WB), jnp.int32)


def _aux_body(x1r, y1r, x2r, y2r, scr, gir, orr, gx1, gy1, gx2, gy2,
              pl_out, pb_out, kp_out):
    gidx = jax.lax.broadcasted_iota(jnp.int32, (G, 1), 0)
    gi = gir[...]
    eq = gi == gidx
    ggx1 = jnp.sum(jnp.where(eq, gx1[...], 0.0), axis=0, keepdims=True)
    ggy1 = jnp.sum(jnp.where(eq, gy1[...], 0.0), axis=0, keepdims=True)
    ggx2 = jnp.sum(jnp.where(eq, gx2[...], 0.0), axis=0, keepdims=True)
    ggy2 = jnp.sum(jnp.where(eq, gy2[...], 0.0), axis=0, keepdims=True)
    bx1 = x1r[...]; by1 = y1r[...]; bx2 = x2r[...]; by2 = y2r[...]
    area_g = (ggx2 - ggx1) * (ggy2 - ggy1)
    area_b = (bx2 - bx1) * (by2 - by1)
    ltx = jnp.maximum(ggx1, bx1)
    lty = jnp.maximum(ggy1, by1)
    rbx = jnp.minimum(ggx2, bx2)
    rby = jnp.minimum(ggy2, by2)
    wx = jnp.clip(rbx - ltx, 0.0, None)
    wy = jnp.clip(rby - lty, 0.0, None)
    inter = wx * wy
    msi = inter / jnp.maximum(area_g + area_b - inter, EPS)
    msi = jnp.clip(msi, EPS, None)
    pl_out[...] = -jnp.log(jnp.clip(1.0 - NMS_THR + msi, None, 1.0)) * scr[...]
    minf = jnp.float32(-jnp.inf)
    msc = jnp.where(eq, scr[...], minf)
    best = jnp.max(msc, axis=1, keepdims=True)
    cand2 = eq & (msc == best)
    oid = orr[...]
    pi = jnp.min(jnp.where(cand2, oid, N), axis=1, keepdims=True)
    sel = cand2 & (oid == pi)
    sx1 = jnp.sum(jnp.where(sel, bx1, 0.0), axis=1, keepdims=True)
    sy1 = jnp.sum(jnp.where(sel, by1, 0.0), axis=1, keepdims=True)
    sx2 = jnp.sum(jnp.where(sel, bx2, 0.0), axis=1, keepdims=True)
    sy2 = jnp.sum(jnp.where(sel, by2, 0.0), axis=1, keepdims=True)
    g_area = (gx2[...] - gx1[...]) * (gy2[...] - gy1[...])
    s_area = (sx2 - sx1) * (sy2 - sy1)
    ltx2 = jnp.maximum(gx1[...], sx1)
    lty2 = jnp.maximum(gy1[...], sy1)
    rbx2 = jnp.minimum(gx2[...], sx2)
    rby2 = jnp.minimum(gy2[...], sy2)
    wx2 = jnp.clip(rbx2 - ltx2, 0.0, None)
    wy2 = jnp.clip(rby2 - lty2, 0.0, None)
    inter2 = wx2 * wy2
    gsel = inter2 / jnp.maximum(g_area + s_area - inter2, EPS)
    has = jnp.any(eq, axis=1, keepdims=True)
    height = gy2[...] - gy1[...]
    kp_out[...] = (has & (height >= MIN_HEIGHT)).astype(jnp.int32)
    pb_out[...] = jnp.where(has, 1.0 - gsel, 0.0)


def _scan_body(gi_sref, pl_sref, m_ref, seen_out, scal_out, acc_ref, accp_ref):
    def init_g(g, c):
        seen_out[0, g] = 0
        return c
    jax.lax.fori_loop(0, G, init_g, 0)
    acc_ref[...] = jnp.zeros((1, W16), jnp.int32)
    accp_ref[...] = jnp.zeros((1, W16), jnp.int32)
    lanes = jax.lax.broadcasted_iota(jnp.int32, (1, W16), 1)

    def _extract(acc, w):
        return jnp.sum(acc * (lanes == w).astype(jnp.int32))

    def body(p, carry):
        word, pcnt, psum, l_cnt, l_pl, l_p = carry
        w = p // 16
        b = p - w * 16
        word = jax.lax.cond(b == 0,
                            lambda: _extract(acc_ref[...], w),
                            lambda: word)
        kept = ((word >> b) & 1) == 0

        def kept_fn():
            gv = gi_sref[0, p]
            counted = seen_out[0, gv]
            seen_out[0, gv] = 1
            acc = acc_ref[...]
            row = m_ref[pl.ds(p, 1), :]
            accp_ref[...] = acc
            acc_new = acc | row
            acc_ref[...] = acc_new
            word2 = _extract(acc_new, w)
            plv = pl_sref[0, p]
            keptc = (counted > 0).astype(jnp.int32)
            return (word2, pcnt + keptc,
                    psum + jnp.where(keptc > 0, plv, 0.0),
                    keptc, plv, p)

        def sup_fn():
            return (word, pcnt, psum, l_cnt, l_pl, l_p)

        return jax.lax.cond(kept, kept_fn, sup_fn)

    _, pcnt, psum, l_cnt, l_pl, l_p = jax.lax.fori_loop(
        0, N, body,
        (jnp.int32(0), jnp.int32(0), jnp.float32(0.0),
         jnp.int32(0), jnp.float32(0.0), jnp.int32(0)))
    accp = accp_ref[...]
    base = lanes * 16
    s1 = jnp.clip((l_p + 1) - base, 0, 16)
    mask_lo = jnp.left_shift(jnp.int32(0xFFFF), s1) & 0xFFFF
    s2 = jnp.clip(N - base, 0, 16)
    mask_hi = jnp.left_shift(jnp.int32(1), s2) - 1
    zeros_alive = jnp.bitwise_not(accp) & (mask_lo & mask_hi)
    l_rem = jnp.max(zeros_alive) > 0
    scal_out[0, 0] = pcnt.astype(jnp.float32)
    scal_out[0, 1] = psum
    scal_out[0, 2] = l_cnt.astype(jnp.float32)
    scal_out[0, 3] = l_rem.astype(jnp.float32)
    scal_out[0, 4] = l_pl


def _pack_call(x1c, y1c, x2c, y2c, x1r, y1r, x2r, y2r):
    col = pl.BlockSpec((TB, 1), lambda i: (i, 0))
    row = pl.BlockSpec((1, NP), lambda i: (0, 0))
    return pl.pallas_call(
        _pack_body,
        grid=(NP // TB,),
        in_specs=[col, col, col, col, row, row, row, row],
        out_specs=pl.BlockSpec((TB, W16), lambda i: (i, 0)),
        out_shape=jax.ShapeDtypeStruct((NP, W16), jnp.int32),
    )(x1c, y1c, x2c, y2c, x1r, y1r, x2r, y2r)


def _aux_call(x1r, y1r, x2r, y2r, scr, gir, orr, gx1, gy1, gx2, gy2):
    return pl.pallas_call(
        _aux_body,
        in_specs=[pl.BlockSpec()] * 11,
        out_specs=[pl.BlockSpec(), pl.BlockSpec(), pl.BlockSpec()],
        out_shape=[
            jax.ShapeDtypeStruct((1, NP), jnp.float32),
            jax.ShapeDtypeStruct((G, 1), jnp.float32),
            jax.ShapeDtypeStruct((G, 1), jnp.int32),
        ],
    )(x1r, y1r, x2r, y2r, scr, gir, orr, gx1, gy1, gx2, gy2)


def _scan_call(gir, plv, m):
    smem = pl.BlockSpec(memory_space=pltpu.SMEM)
    return pl.pallas_call(
        _scan_body,
        in_specs=[smem, smem, pl.BlockSpec()],
        out_specs=[smem, smem],
        out_shape=[
            jax.ShapeDtypeStruct((1, G), jnp.int32),
            jax.ShapeDtypeStruct((1, 8), jnp.float32),
        ],
        scratch_shapes=[
            pltpu.VMEM((1, W16), jnp.int32),
            pltpu.VMEM((1, W16), jnp.int32),
        ],
    )(gir, plv, m)


def kernel(gt_inds, anchor_gt_inds, gt_bboxes, proposal_list):
    del gt_inds
    gi0 = anchor_gt_inds[0].astype(jnp.int32)
    props = proposal_list[0]
    gtb = gt_bboxes[0]
    scores = props[:, 4]
    order = jnp.argsort(scores)[::-1].astype(jnp.int32)
    bs = props[order]
    gis = gi0[order]
    pad = NP - N
    x1 = jnp.pad(bs[:, 0], (0, pad))
    y1 = jnp.pad(bs[:, 1], (0, pad))
    x2 = jnp.pad(bs[:, 2], (0, pad))
    y2 = jnp.pad(bs[:, 3], (0, pad))
    sc = jnp.pad(bs[:, 4], (0, pad))
    gip = jnp.pad(gis, (0, pad), constant_values=-1)
    orp = jnp.pad(order, (0, pad), constant_values=N)
    x1c, y1c, x2c, y2c = (a[:, None] for a in (x1, y1, x2, y2))
    x1r, y1r, x2r, y2r, scr = (a[None, :] for a in (x1, y1, x2, y2, sc))
    gir = gip[None, :]
    orr = orp[None, :]
    gx1, gy1, gx2, gy2 = (gtb[:, k][:, None] for k in range(4))

    m = _pack_call(x1c, y1c, x2c, y2c, x1r, y1r, x2r, y2r)
    plv, pb, kp = _aux_call(x1r, y1r, x2r, y2r, scr, gir, orr, gx1, gy1, gx2, gy2)
    seen, scal = _scan_call(gir, plv, m)

    pull_cnt = scal[0, 0]
    psum = scal[0, 1]
    l_cnt = scal[0, 2]
    l_rem = scal[0, 3]
    l_pl = scal[0, 4]
    total_pull = psum - jnp.where((l_cnt > 0) & (l_rem == 0), l_pl, 0.0)
    pull_loss = total_pull / (pull_cnt + EPS)
    keepg = (kp[:, 0] > 0) & (seen[0, :] == 0)
    total_push = jnp.sum(jnp.where(keepg, pb[:, 0], 0.0))
    push_cnt = jnp.sum(keepg)
    push_loss = total_push / (push_cnt + EPS)
    return jnp.stack([push_loss, pull_loss])


# speedup vs baseline: 371.4418x; 1.5021x over previous
import jax
import jax.numpy as jnp
from jax.experimental import pallas as pl
from jax.experimental.pallas import tpu as pltpu

NMS_THR = 0.5
MIN_HEIGHT = 50.0
EPS = 1e-6
N = 5000
G = 128
NP = 5120
TB = 512
W16 = NP // 16
WB = TB // 16


def _pack_body(x1c, y1c, x2c, y2c, x1r, y1r, x2r, y2r, m_ref):
    ib = pl.program_id(0)
    x1i = x1c[...]; y1i = y1c[...]; x2i = x2c[...]; y2i = y2c[...]
    a_i = (x2i - x1i) * (y2i - y1i)
    row_iota = jax.lax.broadcasted_iota(jnp.int32, (TB, TB), 0)
    col_iota = jax.lax.broadcasted_iota(jnp.int32, (TB, TB), 1)
    ci = jax.lax.broadcasted_iota(jnp.int32, (TB, WB), 0)
    wi = jax.lax.broadcasted_iota(jnp.int32, (TB, WB), 1)
    pmat = jnp.where((ci // 16) == wi,
                     jnp.left_shift(jnp.int32(1), ci % 16), 0).astype(jnp.float32)

    for jb in range(NP // TB):
        sl = slice(jb * WB, (jb + 1) * WB)

        @pl.when(jb >= ib)
        def _(jb=jb, sl=sl):
            cs = slice(jb * TB, (jb + 1) * TB)
            x1j = x1r[:, cs]; y1j = y1r[:, cs]
            x2j = x2r[:, cs]; y2j = y2r[:, cs]
            a_j = (x2j - x1j) * (y2j - y1j)
            ltx = jnp.maximum(x1i, x1j)
            lty = jnp.maximum(y1i, y1j)
            rbx = jnp.minimum(x2i, x2j)
            rby = jnp.minimum(y2i, y2j)
            wx = jnp.clip(rbx - ltx, 0.0, None)
            wy = jnp.clip(rby - lty, 0.0, None)
            inter = wx * wy
            union = a_i + a_j - inter
            iou = inter / jnp.maximum(union, EPS)
            row_i = ib * TB + row_iota
            col_j = jb * TB + col_iota
            over = (iou > NMS_THR) & (col_j > row_i) & (col_j < N)
            bits = jax.lax.dot_general(over.astype(jnp.float32), pmat,
                                       (((1,), (0,)), ((), ())),
                                       preferred_element_type=jnp.float32)
            m_ref[:, sl] = bits.astype(jnp.int32)

        @pl.when(jb < ib)
        def _(sl=sl):
            m_ref[:, sl] = jnp.zeros((TB, WB), jnp.int32)


def _aux_body(x1r, y1r, x2r, y2r, scr, gir, orr, gx1, gy1, gx2, gy2,
              pl_out, pb_out, kp_out):
    gidx = jax.lax.broadcasted_iota(jnp.int32, (G, 1), 0)
    gi = gir[...]
    eq = gi == gidx
    ggx1 = jnp.sum(jnp.where(eq, gx1[...], 0.0), axis=0, keepdims=True)
    ggy1 = jnp.sum(jnp.where(eq, gy1[...], 0.0), axis=0, keepdims=True)
    ggx2 = jnp.sum(jnp.where(eq, gx2[...], 0.0), axis=0, keepdims=True)
    ggy2 = jnp.sum(jnp.where(eq, gy2[...], 0.0), axis=0, keepdims=True)
    bx1 = x1r[...]; by1 = y1r[...]; bx2 = x2r[...]; by2 = y2r[...]
    area_g = (ggx2 - ggx1) * (ggy2 - ggy1)
    area_b = (bx2 - bx1) * (by2 - by1)
    ltx = jnp.maximum(ggx1, bx1)
    lty = jnp.maximum(ggy1, by1)
    rbx = jnp.minimum(ggx2, bx2)
    rby = jnp.minimum(ggy2, by2)
    wx = jnp.clip(rbx - ltx, 0.0, None)
    wy = jnp.clip(rby - lty, 0.0, None)
    inter = wx * wy
    msi = inter / jnp.maximum(area_g + area_b - inter, EPS)
    msi = jnp.clip(msi, EPS, None)
    pl_out[...] = -jnp.log(jnp.clip(1.0 - NMS_THR + msi, None, 1.0)) * scr[...]
    minf = jnp.float32(-jnp.inf)
    msc = jnp.where(eq, scr[...], minf)
    best = jnp.max(msc, axis=1, keepdims=True)
    cand2 = eq & (msc == best)
    oid = orr[...]
    pi = jnp.min(jnp.where(cand2, oid, N), axis=1, keepdims=True)
    sel = cand2 & (oid == pi)
    sx1 = jnp.sum(jnp.where(sel, bx1, 0.0), axis=1, keepdims=True)
    sy1 = jnp.sum(jnp.where(sel, by1, 0.0), axis=1, keepdims=True)
    sx2 = jnp.sum(jnp.where(sel, bx2, 0.0), axis=1, keepdims=True)
    sy2 = jnp.sum(jnp.where(sel, by2, 0.0), axis=1, keepdims=True)
    g_area = (gx2[...] - gx1[...]) * (gy2[...] - gy1[...])
    s_area = (sx2 - sx1) * (sy2 - sy1)
    ltx2 = jnp.maximum(gx1[...], sx1)
    lty2 = jnp.maximum(gy1[...], sy1)
    rbx2 = jnp.minimum(gx2[...], sx2)
    rby2 = jnp.minimum(gy2[...], sy2)
    wx2 = jnp.clip(rbx2 - ltx2, 0.0, None)
    wy2 = jnp.clip(rby2 - lty2, 0.0, None)
    inter2 = wx2 * wy2
    gsel = inter2 / jnp.maximum(g_area + s_area - inter2, EPS)
    has = jnp.any(eq, axis=1, keepdims=True)
    height = gy2[...] - gy1[...]
    kp_out[...] = (has & (height >= MIN_HEIGHT)).astype(jnp.int32)
    pb_out[...] = jnp.where(has, 1.0 - gsel, 0.0)


def _scan_body(gi_sref, pl_sref, m_ref, seen_out, scal_out, acc_ref, accp_ref):
    def init_g(g, c):
        seen_out[0, g] = 0
        return c
    jax.lax.fori_loop(0, G, init_g, 0)
    lanes = jax.lax.broadcasted_iota(jnp.int32, (1, W16), 1)
    NW = (N + 15) // 16
    base = lanes * 16
    svalid = jnp.clip(N - base, 0, 16)
    acc_ref[...] = jnp.bitwise_not(
        jnp.left_shift(jnp.int32(1), svalid) - 1) & 0xFFFF
    accp_ref[...] = jnp.zeros((1, W16), jnp.int32)

    def _extract(acc, w):
        return jnp.sum(acc * (lanes == w).astype(jnp.int32))

    def wcond(st):
        return st[0] < NW

    def wbody(st):
        w, word, pcnt, psum, l_cnt, l_pl, l_p = st
        free = jnp.bitwise_not(word) & 0xFFFF

        def advance():
            return (w + 1, _extract(acc_ref[...], w + 1),
                    pcnt, psum, l_cnt, l_pl, l_p)

        def pick():
            low = free & (-free)
            b = (jax.lax.bitcast_convert_type(
                low.astype(jnp.float32), jnp.int32) >> 23) - 127
            p = w * 16 + b
            gv = gi_sref[0, p]
            counted = seen_out[0, gv]
            seen_out[0, gv] = 1
            acc = acc_ref[...]
            row = m_ref[pl.ds(p, 1), :]
            accp_ref[...] = acc
            acc_new = acc | row
            acc_ref[...] = acc_new
            word2 = _extract(acc_new, w) | word | low
            plv = pl_sref[0, p]
            keptc = (counted > 0).astype(jnp.int32)
            return (w, word2, pcnt + keptc,
                    psum + jnp.where(keptc > 0, plv, 0.0),
                    keptc, plv, p)

        return jax.lax.cond(free == 0, advance, pick)

    _, _, pcnt, psum, l_cnt, l_pl, l_p = jax.lax.while_loop(
        wcond, wbody,
        (jnp.int32(0), _extract(acc_ref[...], 0),
         jnp.int32(0), jnp.float32(0.0),
         jnp.int32(0), jnp.float32(0.0), jnp.int32(0)))
    accp = accp_ref[...]
    base = lanes * 16
    s1 = jnp.clip((l_p + 1) - base, 0, 16)
    mask_lo = jnp.left_shift(jnp.int32(0xFFFF), s1) & 0xFFFF
    s2 = jnp.clip(N - base, 0, 16)
    mask_hi = jnp.left_shift(jnp.int32(1), s2) - 1
    zeros_alive = jnp.bitwise_not(accp) & (mask_lo & mask_hi)
    l_rem = jnp.max(zeros_alive) > 0
    scal_out[0, 0] = pcnt.astype(jnp.float32)
    scal_out[0, 1] = psum
    scal_out[0, 2] = l_cnt.astype(jnp.float32)
    scal_out[0, 3] = l_rem.astype(jnp.float32)
    scal_out[0, 4] = l_pl


def _pack_call(x1c, y1c, x2c, y2c, x1r, y1r, x2r, y2r):
    col = pl.BlockSpec((TB, 1), lambda i: (i, 0))
    row = pl.BlockSpec((1, NP), lambda i: (0, 0))
    return pl.pallas_call(
        _pack_body,
        grid=(NP // TB,),
        in_specs=[col, col, col, col, row, row, row, row],
        out_specs=pl.BlockSpec((TB, W16), lambda i: (i, 0)),
        out_shape=jax.ShapeDtypeStruct((NP, W16), jnp.int32),
    )(x1c, y1c, x2c, y2c, x1r, y1r, x2r, y2r)


def _aux_call(x1r, y1r, x2r, y2r, scr, gir, orr, gx1, gy1, gx2, gy2):
    return pl.pallas_call(
        _aux_body,
        in_specs=[pl.BlockSpec()] * 11,
        out_specs=[pl.BlockSpec(), pl.BlockSpec(), pl.BlockSpec()],
        out_shape=[
            jax.ShapeDtypeStruct((1, NP), jnp.float32),
            jax.ShapeDtypeStruct((G, 1), jnp.float32),
            jax.ShapeDtypeStruct((G, 1), jnp.int32),
        ],
    )(x1r, y1r, x2r, y2r, scr, gir, orr, gx1, gy1, gx2, gy2)


def _scan_call(gir, plv, m):
    smem = pl.BlockSpec(memory_space=pltpu.SMEM)
    return pl.pallas_call(
        _scan_body,
        in_specs=[smem, smem, pl.BlockSpec()],
        out_specs=[smem, smem],
        out_shape=[
            jax.ShapeDtypeStruct((1, G), jnp.int32),
            jax.ShapeDtypeStruct((1, 8), jnp.float32),
        ],
        scratch_shapes=[
            pltpu.VMEM((1, W16), jnp.int32),
            pltpu.VMEM((1, W16), jnp.int32),
        ],
    )(gir, plv, m)


def kernel(gt_inds, anchor_gt_inds, gt_bboxes, proposal_list):
    del gt_inds
    gi0 = anchor_gt_inds[0].astype(jnp.int32)
    props = proposal_list[0]
    gtb = gt_bboxes[0]
    scores = props[:, 4]
    order = jnp.argsort(scores)[::-1].astype(jnp.int32)
    bs = props[order]
    gis = gi0[order]
    pad = NP - N
    x1 = jnp.pad(bs[:, 0], (0, pad))
    y1 = jnp.pad(bs[:, 1], (0, pad))
    x2 = jnp.pad(bs[:, 2], (0, pad))
    y2 = jnp.pad(bs[:, 3], (0, pad))
    sc = jnp.pad(bs[:, 4], (0, pad))
    gip = jnp.pad(gis, (0, pad), constant_values=-1)
    orp = jnp.pad(order, (0, pad), constant_values=N)
    x1c, y1c, x2c, y2c = (a[:, None] for a in (x1, y1, x2, y2))
    x1r, y1r, x2r, y2r, scr = (a[None, :] for a in (x1, y1, x2, y2, sc))
    gir = gip[None, :]
    orr = orp[None, :]
    gx1, gy1, gx2, gy2 = (gtb[:, k][:, None] for k in range(4))

    m = _pack_call(x1c, y1c, x2c, y2c, x1r, y1r, x2r, y2r)
    plv, pb, kp = _aux_call(x1r, y1r, x2r, y2r, scr, gir, orr, gx1, gy1, gx2, gy2)
    seen, scal = _scan_call(gir, plv, m)

    pull_cnt = scal[0, 0]
    psum = scal[0, 1]
    l_cnt = scal[0, 2]
    l_rem = scal[0, 3]
    l_pl = scal[0, 4]
    total_pull = psum - jnp.where((l_cnt > 0) & (l_rem == 0), l_pl, 0.0)
    pull_loss = total_pull / (pull_cnt + EPS)
    keepg = (kp[:, 0] > 0) & (seen[0, :] == 0)
    total_push = jnp.sum(jnp.where(keepg, pb[:, 0], 0.0))
    push_cnt = jnp.sum(keepg)
    push_loss = total_push / (push_cnt + EPS)
    return jnp.stack([push_loss, pull_loss])


# speedup vs baseline: 387.8190x; 1.0441x over previous
import jax
import jax.numpy as jnp
from jax.experimental import pallas as pl
from jax.experimental.pallas import tpu as pltpu

NMS_THR = 0.5
MIN_HEIGHT = 50.0
EPS = 1e-6
N = 5000
G = 128
NP = 5120
TB = 512
W16 = NP // 16
WB = TB // 16


def _pack_body(x1c, y1c, x2c, y2c, x1r, y1r, x2r, y2r, m_ref):
    ib = pl.program_id(0)
    x1i = x1c[...]; y1i = y1c[...]; x2i = x2c[...]; y2i = y2c[...]
    a_i = (x2i - x1i) * (y2i - y1i)
    row_iota = jax.lax.broadcasted_iota(jnp.int32, (TB, TB), 0)
    col_iota = jax.lax.broadcasted_iota(jnp.int32, (TB, TB), 1)
    ci = jax.lax.broadcasted_iota(jnp.int32, (TB, WB), 0)
    wi = jax.lax.broadcasted_iota(jnp.int32, (TB, WB), 1)
    pmat = jnp.where((ci // 16) == wi,
                     jnp.left_shift(jnp.int32(1), ci % 16), 0).astype(jnp.bfloat16)

    for jb in range(NP // TB):
        sl = slice(jb * WB, (jb + 1) * WB)

        @pl.when(jb >= ib)
        def _(jb=jb, sl=sl):
            cs = slice(jb * TB, (jb + 1) * TB)
            x1j = x1r[:, cs]; y1j = y1r[:, cs]
            x2j = x2r[:, cs]; y2j = y2r[:, cs]
            a_j = (x2j - x1j) * (y2j - y1j)
            ltx = jnp.maximum(x1i, x1j)
            lty = jnp.maximum(y1i, y1j)
            rbx = jnp.minimum(x2i, x2j)
            rby = jnp.minimum(y2i, y2j)
            wx = jnp.clip(rbx - ltx, 0.0, None)
            wy = jnp.clip(rby - lty, 0.0, None)
            inter = wx * wy
            union = a_i + a_j - inter
            iou = inter / jnp.maximum(union, EPS)
            row_i = ib * TB + row_iota
            col_j = jb * TB + col_iota
            over = (iou > NMS_THR) & (col_j > row_i) & (col_j < N)
            bits = jax.lax.dot_general(over.astype(jnp.bfloat16), pmat,
                                       (((1,), (0,)), ((), ())),
                                       preferred_element_type=jnp.float32)
            m_ref[:, sl] = bits.astype(jnp.int32)

        @pl.when(jb < ib)
        def _(sl=sl):
            m_ref[:, sl] = jnp.zeros((TB, WB), jnp.int32)


def _aux_body(x1r, y1r, x2r, y2r, scr, gir, orr, gx1, gy1, gx2, gy2,
              pl_out, pb_out, kp_out):
    gidx = jax.lax.broadcasted_iota(jnp.int32, (G, 1), 0)
    gi = gir[...]
    eq = gi == gidx
    ggx1 = jnp.sum(jnp.where(eq, gx1[...], 0.0), axis=0, keepdims=True)
    ggy1 = jnp.sum(jnp.where(eq, gy1[...], 0.0), axis=0, keepdims=True)
    ggx2 = jnp.sum(jnp.where(eq, gx2[...], 0.0), axis=0, keepdims=True)
    ggy2 = jnp.sum(jnp.where(eq, gy2[...], 0.0), axis=0, keepdims=True)
    bx1 = x1r[...]; by1 = y1r[...]; bx2 = x2r[...]; by2 = y2r[...]
    area_g = (ggx2 - ggx1) * (ggy2 - ggy1)
    area_b = (bx2 - bx1) * (by2 - by1)
    ltx = jnp.maximum(ggx1, bx1)
    lty = jnp.maximum(ggy1, by1)
    rbx = jnp.minimum(ggx2, bx2)
    rby = jnp.minimum(ggy2, by2)
    wx = jnp.clip(rbx - ltx, 0.0, None)
    wy = jnp.clip(rby - lty, 0.0, None)
    inter = wx * wy
    msi = inter / jnp.maximum(area_g + area_b - inter, EPS)
    msi = jnp.clip(msi, EPS, None)
    pl_out[...] = -jnp.log(jnp.clip(1.0 - NMS_THR + msi, None, 1.0)) * scr[...]
    minf = jnp.float32(-jnp.inf)
    msc = jnp.where(eq, scr[...], minf)
    best = jnp.max(msc, axis=1, keepdims=True)
    cand2 = eq & (msc == best)
    oid = orr[...]
    pi = jnp.min(jnp.where(cand2, oid, N), axis=1, keepdims=True)
    sel = cand2 & (oid == pi)
    sx1 = jnp.sum(jnp.where(sel, bx1, 0.0), axis=1, keepdims=True)
    sy1 = jnp.sum(jnp.where(sel, by1, 0.0), axis=1, keepdims=True)
    sx2 = jnp.sum(jnp.where(sel, bx2, 0.0), axis=1, keepdims=True)
    sy2 = jnp.sum(jnp.where(sel, by2, 0.0), axis=1, keepdims=True)
    g_area = (gx2[...] - gx1[...]) * (gy2[...] - gy1[...])
    s_area = (sx2 - sx1) * (sy2 - sy1)
    ltx2 = jnp.maximum(gx1[...], sx1)
    lty2 = jnp.maximum(gy1[...], sy1)
    rbx2 = jnp.minimum(gx2[...], sx2)
    rby2 = jnp.minimum(gy2[...], sy2)
    wx2 = jnp.clip(rbx2 - ltx2, 0.0, None)
    wy2 = jnp.clip(rby2 - lty2, 0.0, None)
    inter2 = wx2 * wy2
    gsel = inter2 / jnp.maximum(g_area + s_area - inter2, EPS)
    has = jnp.any(eq, axis=1, keepdims=True)
    height = gy2[...] - gy1[...]
    kp_out[...] = (has & (height >= MIN_HEIGHT)).astype(jnp.int32)
    pb_out[...] = jnp.where(has, 1.0 - gsel, 0.0)


def _scan_body(gi_sref, pl_sref, kp_sref, pb_sref, m_ref, out_sref,
               acc_ref, accp_ref, seen_scr):
    def init_g(g, c):
        seen_scr[0, g] = 0
        return c
    jax.lax.fori_loop(0, G, init_g, 0)
    lanes = jax.lax.broadcasted_iota(jnp.int32, (1, W16), 1)
    NW = (N + 15) // 16
    base = lanes * 16
    svalid = jnp.clip(N - base, 0, 16)
    acc_ref[...] = jnp.bitwise_not(
        jnp.left_shift(jnp.int32(1), svalid) - 1) & 0xFFFF
    accp_ref[...] = jnp.zeros((1, W16), jnp.int32)

    def _extract(acc, w):
        return jnp.sum(acc * (lanes == w).astype(jnp.int32))

    def wcond(st):
        return st[0] < NW

    def wbody(st):
        w, word, pcnt, psum, l_cnt, l_pl, l_p = st
        free = jnp.bitwise_not(word) & 0xFFFF

        def advance():
            return (w + 1, _extract(acc_ref[...], w + 1),
                    pcnt, psum, l_cnt, l_pl, l_p)

        def pick():
            low = free & (-free)
            b = (jax.lax.bitcast_convert_type(
                low.astype(jnp.float32), jnp.int32) >> 23) - 127
            p = w * 16 + b
            gv = gi_sref[0, p]
            counted = seen_scr[0, gv]
            seen_scr[0, gv] = 1
            acc = acc_ref[...]
            row = m_ref[pl.ds(p, 1), :]
            accp_ref[...] = acc
            acc_new = acc | row
            acc_ref[...] = acc_new
            word2 = _extract(acc_new, w) | word | low
            plv = pl_sref[0, p]
            keptc = (counted > 0).astype(jnp.int32)
            return (w, word2, pcnt + keptc,
                    psum + jnp.where(keptc > 0, plv, 0.0),
                    keptc, plv, p)

        return jax.lax.cond(free == 0, advance, pick)

    _, _, pcnt, psum, l_cnt, l_pl, l_p = jax.lax.while_loop(
        wcond, wbody,
        (jnp.int32(0), _extract(acc_ref[...], 0),
         jnp.int32(0), jnp.float32(0.0),
         jnp.int32(0), jnp.float32(0.0), jnp.int32(0)))
    accp = accp_ref[...]
    base = lanes * 16
    s1 = jnp.clip((l_p + 1) - base, 0, 16)
    mask_lo = jnp.left_shift(jnp.int32(0xFFFF), s1) & 0xFFFF
    s2 = jnp.clip(N - base, 0, 16)
    mask_hi = jnp.left_shift(jnp.int32(1), s2) - 1
    zeros_alive = jnp.bitwise_not(accp) & (mask_lo & mask_hi)
    l_rem = jnp.max(zeros_alive) > 0
    total_pull = psum - jnp.where((l_cnt > 0) & (~l_rem), l_pl, 0.0)
    pull_loss = total_pull / (pcnt.astype(jnp.float32) + EPS)

    def pg(g, c):
        tp, pc = c
        k = (kp_sref[g, 0] > 0) & (seen_scr[0, g] == 0)
        tp = tp + jnp.where(k, pb_sref[g, 0], 0.0)
        pc = pc + jnp.where(k, 1, 0).astype(jnp.int32)
        return tp, pc

    total_push, push_cnt = jax.lax.fori_loop(
        0, G, pg, (jnp.float32(0.0), jnp.int32(0)))
    push_loss = total_push / (push_cnt.astype(jnp.float32) + EPS)
    out_sref[0, 0] = push_loss
    out_sref[0, 1] = pull_loss


def _pack_call(x1c, y1c, x2c, y2c, x1r, y1r, x2r, y2r):
    col = pl.BlockSpec((TB, 1), lambda i: (i, 0))
    row = pl.BlockSpec((1, NP), lambda i: (0, 0))
    return pl.pallas_call(
        _pack_body,
        grid=(NP // TB,),
        in_specs=[col, col, col, col, row, row, row, row],
        out_specs=pl.BlockSpec((TB, W16), lambda i: (i, 0)),
        out_shape=jax.ShapeDtypeStruct((NP, W16), jnp.int32),
    )(x1c, y1c, x2c, y2c, x1r, y1r, x2r, y2r)


def _aux_call(x1r, y1r, x2r, y2r, scr, gir, orr, gx1, gy1, gx2, gy2):
    return pl.pallas_call(
        _aux_body,
        in_specs=[pl.BlockSpec()] * 11,
        out_specs=[pl.BlockSpec(), pl.BlockSpec(), pl.BlockSpec()],
        out_shape=[
            jax.ShapeDtypeStruct((1, NP), jnp.float32),
            jax.ShapeDtypeStruct((G, 1), jnp.float32),
            jax.ShapeDtypeStruct((G, 1), jnp.int32),
        ],
    )(x1r, y1r, x2r, y2r, scr, gir, orr, gx1, gy1, gx2, gy2)


def _scan_call(gir, plv, kp, pb, m):
    smem = pl.BlockSpec(memory_space=pltpu.SMEM)
    return pl.pallas_call(
        _scan_body,
        in_specs=[smem, smem, smem, smem, pl.BlockSpec()],
        out_specs=smem,
        out_shape=jax.ShapeDtypeStruct((1, 2), jnp.float32),
        scratch_shapes=[
            pltpu.VMEM((1, W16), jnp.int32),
            pltpu.VMEM((1, W16), jnp.int32),
            pltpu.SMEM((1, G), jnp.int32),
        ],
    )(gir, plv, kp, pb, m)


def kernel(gt_inds, anchor_gt_inds, gt_bboxes, proposal_list):
    del gt_inds
    gi0 = anchor_gt_inds[0].astype(jnp.int32)
    props = proposal_list[0]
    gtb = gt_bboxes[0]
    scores = props[:, 4]
    order = jnp.argsort(scores)[::-1].astype(jnp.int32)
    bs = props[order]
    gis = gi0[order]
    pad = NP - N
    x1 = jnp.pad(bs[:, 0], (0, pad))
    y1 = jnp.pad(bs[:, 1], (0, pad))
    x2 = jnp.pad(bs[:, 2], (0, pad))
    y2 = jnp.pad(bs[:, 3], (0, pad))
    sc = jnp.pad(bs[:, 4], (0, pad))
    gip = jnp.pad(gis, (0, pad), constant_values=-1)
    orp = jnp.pad(order, (0, pad), constant_values=N)
    x1c, y1c, x2c, y2c = (a[:, None] for a in (x1, y1, x2, y2))
    x1r, y1r, x2r, y2r, scr = (a[None, :] for a in (x1, y1, x2, y2, sc))
    gir = gip[None, :]
    orr = orp[None, :]
    gx1, gy1, gx2, gy2 = (gtb[:, k][:, None] for k in range(4))

    m = _pack_call(x1c, y1c, x2c, y2c, x1r, y1r, x2r, y2r)
    plv, pb, kp = _aux_call(x1r, y1r, x2r, y2r, scr, gir, orr, gx1, gy1, gx2, gy2)
    out = _scan_call(gir, plv, kp, pb, m)
    return out[0, :]


# speedup vs baseline: 430.8728x; 1.1110x over previous
import functools

import jax
import jax.numpy as jnp
from jax.experimental import pallas as pl
from jax.experimental.pallas import tpu as pltpu
from jax.experimental.pallas import tpu_sc as plsc

NMS_THR = 0.5
MIN_HEIGHT = 50.0
EPS = 1e-6
N = 5000
G = 128
NP = 5120
TB = 512
W16 = NP // 16
WB = TB // 16


def _pack_body(x1c, y1c, x2c, y2c, x1r, y1r, x2r, y2r, m_ref):
    ib = pl.program_id(0)
    x1i = x1c[...]; y1i = y1c[...]; x2i = x2c[...]; y2i = y2c[...]
    a_i = (x2i - x1i) * (y2i - y1i)
    row_iota = jax.lax.broadcasted_iota(jnp.int32, (TB, TB), 0)
    col_iota = jax.lax.broadcasted_iota(jnp.int32, (TB, TB), 1)
    ci = jax.lax.broadcasted_iota(jnp.int32, (TB, WB), 0)
    wi = jax.lax.broadcasted_iota(jnp.int32, (TB, WB), 1)
    pmat = jnp.where((ci // 16) == wi,
                     jnp.left_shift(jnp.int32(1), ci % 16), 0).astype(jnp.bfloat16)

    for jb in range(NP // TB):
        sl = slice(jb * WB, (jb + 1) * WB)

        @pl.when(jb >= ib)
        def _(jb=jb, sl=sl):
            cs = slice(jb * TB, (jb + 1) * TB)
            x1j = x1r[:, cs]; y1j = y1r[:, cs]
            x2j = x2r[:, cs]; y2j = y2r[:, cs]
            a_j = (x2j - x1j) * (y2j - y1j)
            ltx = jnp.maximum(x1i, x1j)
            lty = jnp.maximum(y1i, y1j)
            rbx = jnp.minimum(x2i, x2j)
            rby = jnp.minimum(y2i, y2j)
            wx = jnp.clip(rbx - ltx, 0.0, None)
            wy = jnp.clip(rby - lty, 0.0, None)
            inter = wx * wy
            union = a_i + a_j - inter
            iou = inter / jnp.maximum(union, EPS)
            row_i = ib * TB + row_iota
            col_j = jb * TB + col_iota
            over = (iou > NMS_THR) & (col_j > row_i) & (col_j < N)
            bits = jax.lax.dot_general(over.astype(jnp.bfloat16), pmat,
                                       (((1,), (0,)), ((), ())),
                                       preferred_element_type=jnp.float32)
            m_ref[:, sl] = bits.astype(jnp.int32)

        @pl.when(jb < ib)
        def _(sl=sl):
            m_ref[:, sl] = jnp.zeros((TB, WB), jnp.int32)


def _aux_body(x1r, y1r, x2r, y2r, scr, gir, orr, gx1, gy1, gx2, gy2,
              pl_out, pb_out, kp_out):
    gidx = jax.lax.broadcasted_iota(jnp.int32, (G, 1), 0)
    gi = gir[...]
    eq = gi == gidx
    ggx1 = jnp.sum(jnp.where(eq, gx1[...], 0.0), axis=0, keepdims=True)
    ggy1 = jnp.sum(jnp.where(eq, gy1[...], 0.0), axis=0, keepdims=True)
    ggx2 = jnp.sum(jnp.where(eq, gx2[...], 0.0), axis=0, keepdims=True)
    ggy2 = jnp.sum(jnp.where(eq, gy2[...], 0.0), axis=0, keepdims=True)
    bx1 = x1r[...]; by1 = y1r[...]; bx2 = x2r[...]; by2 = y2r[...]
    area_g = (ggx2 - ggx1) * (ggy2 - ggy1)
    area_b = (bx2 - bx1) * (by2 - by1)
    ltx = jnp.maximum(ggx1, bx1)
    lty = jnp.maximum(ggy1, by1)
    rbx = jnp.minimum(ggx2, bx2)
    rby = jnp.minimum(ggy2, by2)
    wx = jnp.clip(rbx - ltx, 0.0, None)
    wy = jnp.clip(rby - lty, 0.0, None)
    inter = wx * wy
    msi = inter / jnp.maximum(area_g + area_b - inter, EPS)
    msi = jnp.clip(msi, EPS, None)
    pl_out[...] = -jnp.log(jnp.clip(1.0 - NMS_THR + msi, None, 1.0)) * scr[...]
    minf = jnp.float32(-jnp.inf)
    msc = jnp.where(eq, scr[...], minf)
    best = jnp.max(msc, axis=1, keepdims=True)
    cand2 = eq & (msc == best)
    oid = orr[...]
    pi = jnp.min(jnp.where(cand2, oid, N), axis=1, keepdims=True)
    sel = cand2 & (oid == pi)
    sx1 = jnp.sum(jnp.where(sel, bx1, 0.0), axis=1, keepdims=True)
    sy1 = jnp.sum(jnp.where(sel, by1, 0.0), axis=1, keepdims=True)
    sx2 = jnp.sum(jnp.where(sel, bx2, 0.0), axis=1, keepdims=True)
    sy2 = jnp.sum(jnp.where(sel, by2, 0.0), axis=1, keepdims=True)
    g_area = (gx2[...] - gx1[...]) * (gy2[...] - gy1[...])
    s_area = (sx2 - sx1) * (sy2 - sy1)
    ltx2 = jnp.maximum(gx1[...], sx1)
    lty2 = jnp.maximum(gy1[...], sy1)
    rbx2 = jnp.minimum(gx2[...], sx2)
    rby2 = jnp.minimum(gy2[...], sy2)
    wx2 = jnp.clip(rbx2 - ltx2, 0.0, None)
    wy2 = jnp.clip(rby2 - lty2, 0.0, None)
    inter2 = wx2 * wy2
    gsel = inter2 / jnp.maximum(g_area + s_area - inter2, EPS)
    has = jnp.any(eq, axis=1, keepdims=True)
    height = gy2[...] - gy1[...]
    kp_out[...] = (has & (height >= MIN_HEIGHT)).astype(jnp.int32)
    pb_out[...] = jnp.where(has, 1.0 - gsel, 0.0)


def _scan_body(gi_sref, pl_sref, kp_sref, pb_sref, m_ref, out_sref,
               acc_ref, accp_ref, seen_scr):
    def init_g(g, c):
        seen_scr[0, g] = 0
        return c
    jax.lax.fori_loop(0, G, init_g, 0)
    lanes = jax.lax.broadcasted_iota(jnp.int32, (1, W16), 1)
    NW = (N + 15) // 16
    base = lanes * 16
    svalid = jnp.clip(N - base, 0, 16)
    acc_ref[...] = jnp.bitwise_not(
        jnp.left_shift(jnp.int32(1), svalid) - 1) & 0xFFFF
    accp_ref[...] = jnp.zeros((1, W16), jnp.int32)

    def _extract(acc, w):
        return jnp.sum(acc * (lanes == w).astype(jnp.int32))

    def wcond(st):
        return st[0] < NW

    def wbody(st):
        w, word, pcnt, psum, l_cnt, l_pl, l_p = st
        free = jnp.bitwise_not(word) & 0xFFFF

        def advance():
            return (w + 1, _extract(acc_ref[...], w + 1),
                    pcnt, psum, l_cnt, l_pl, l_p)

        def pick():
            low = free & (-free)
            b = (jax.lax.bitcast_convert_type(
                low.astype(jnp.float32), jnp.int32) >> 23) - 127
            p = w * 16 + b
            gv = gi_sref[0, p]
            counted = seen_scr[0, gv]
            seen_scr[0, gv] = 1
            acc = acc_ref[...]
            row = m_ref[pl.ds(p, 1), :]
            accp_ref[...] = acc
            acc_new = acc | row
            acc_ref[...] = acc_new
            word2 = _extract(acc_new, w) | word | low
            plv = pl_sref[0, p]
            keptc = (counted > 0).astype(jnp.int32)
            return (w, word2, pcnt + keptc,
                    psum + jnp.where(keptc > 0, plv, 0.0),
                    keptc, plv, p)

        return jax.lax.cond(free == 0, advance, pick)

    _, _, pcnt, psum, l_cnt, l_pl, l_p = jax.lax.while_loop(
        wcond, wbody,
        (jnp.int32(0), _extract(acc_ref[...], 0),
         jnp.int32(0), jnp.float32(0.0),
         jnp.int32(0), jnp.float32(0.0), jnp.int32(0)))
    accp = accp_ref[...]
    base = lanes * 16
    s1 = jnp.clip((l_p + 1) - base, 0, 16)
    mask_lo = jnp.left_shift(jnp.int32(0xFFFF), s1) & 0xFFFF
    s2 = jnp.clip(N - base, 0, 16)
    mask_hi = jnp.left_shift(jnp.int32(1), s2) - 1
    zeros_alive = jnp.bitwise_not(accp) & (mask_lo & mask_hi)
    l_rem = jnp.max(zeros_alive) > 0
    total_pull = psum - jnp.where((l_cnt > 0) & (~l_rem), l_pl, 0.0)
    pull_loss = total_pull / (pcnt.astype(jnp.float32) + EPS)

    def pg(g, c):
        tp, pc = c
        k = (kp_sref[g, 0] > 0) & (seen_scr[0, g] == 0)
        tp = tp + jnp.where(k, pb_sref[g, 0], 0.0)
        pc = pc + jnp.where(k, 1, 0).astype(jnp.int32)
        return tp, pc

    total_push, push_cnt = jax.lax.fori_loop(
        0, G, pg, (jnp.float32(0.0), jnp.int32(0)))
    push_loss = total_push / (push_cnt.astype(jnp.float32) + EPS)
    out_sref[0, 0] = push_loss
    out_sref[0, 1] = pull_loss


_SC_WORKERS = 32
_BPW = NP // _SC_WORKERS


def _sc_gather_call(c0, c1, c2, c3, c4, c5, idx):
    mesh = plsc.VectorSubcoreMesh(core_axis_name="c", subcore_axis_name="s")
    vec = jax.ShapeDtypeStruct((NP,), jnp.float32)

    @functools.partial(
        pl.kernel, mesh=mesh,
        out_type=[vec] * 6,
        scratch_types=[
            pltpu.VMEM((_BPW,), jnp.int32),
            pltpu.VMEM((_BPW,), jnp.float32),
            pltpu.SemaphoreType.DMA,
        ],
    )
    def k(s0, s1, s2, s3, s4, s5, idx_hbm, o0, o1, o2, o3, o4, o5,
          idx_v, col_v, sem):
        wid = jax.lax.axis_index("s") * 2 + jax.lax.axis_index("c")
        base = wid * _BPW
        pltpu.sync_copy(idx_hbm.at[pl.ds(base, _BPW)], idx_v)
        for src, dst in ((s0, o0), (s1, o1), (s2, o2),
                         (s3, o3), (s4, o4), (s5, o5)):
            pltpu.async_copy(src.at[idx_v], col_v, sem).wait()
            pltpu.sync_copy(col_v, dst.at[pl.ds(base, _BPW)])

    return k(c0, c1, c2, c3, c4, c5, idx)


def _pack_call(x1c, y1c, x2c, y2c, x1r, y1r, x2r, y2r):
    col = pl.BlockSpec((TB, 1), lambda i: (i, 0))
    row = pl.BlockSpec((1, NP), lambda i: (0, 0))
    return pl.pallas_call(
        _pack_body,
        grid=(NP // TB,),
        in_specs=[col, col, col, col, row, row, row, row],
        out_specs=pl.BlockSpec((TB, W16), lambda i: (i, 0)),
        out_shape=jax.ShapeDtypeStruct((NP, W16), jnp.int32),
    )(x1c, y1c, x2c, y2c, x1r, y1r, x2r, y2r)


def _aux_call(x1r, y1r, x2r, y2r, scr, gir, orr, gx1, gy1, gx2, gy2):
    return pl.pallas_call(
        _aux_body,
        in_specs=[pl.BlockSpec()] * 11,
        out_specs=[pl.BlockSpec(), pl.BlockSpec(), pl.BlockSpec()],
        out_shape=[
            jax.ShapeDtypeStruct((1, NP), jnp.float32),
            jax.ShapeDtypeStruct((G, 1), jnp.float32),
            jax.ShapeDtypeStruct((G, 1), jnp.int32),
        ],
    )(x1r, y1r, x2r, y2r, scr, gir, orr, gx1, gy1, gx2, gy2)


def _scan_call(gir, plv, kp, pb, m):
    smem = pl.BlockSpec(memory_space=pltpu.SMEM)
    return pl.pallas_call(
        _scan_body,
        in_specs=[smem, smem, smem, smem, pl.BlockSpec()],
        out_specs=smem,
        out_shape=jax.ShapeDtypeStruct((1, 2), jnp.float32),
        scratch_shapes=[
            pltpu.VMEM((1, W16), jnp.int32),
            pltpu.VMEM((1, W16), jnp.int32),
            pltpu.SMEM((1, G), jnp.int32),
        ],
    )(gir, plv, kp, pb, m)


def kernel(gt_inds, anchor_gt_inds, gt_bboxes, proposal_list):
    del gt_inds
    gi0 = anchor_gt_inds[0].astype(jnp.int32)
    props = proposal_list[0]
    gtb = gt_bboxes[0]
    scores = props[:, 4]
    order = jnp.argsort(scores)[::-1].astype(jnp.int32)
    pad = NP - N
    x1u = jnp.pad(props[:, 0], (0, pad))
    y1u = jnp.pad(props[:, 1], (0, pad))
    x2u = jnp.pad(props[:, 2], (0, pad))
    y2u = jnp.pad(props[:, 3], (0, pad))
    scu = jnp.pad(props[:, 4], (0, pad))
    giu = jnp.pad(gi0.astype(jnp.float32), (0, pad), constant_values=-1.0)
    orp = jnp.pad(order, (0, pad), constant_values=N)
    x1, y1, x2, y2, sc, gif = _sc_gather_call(x1u, y1u, x2u, y2u, scu, giu, orp)
    gip = gif.astype(jnp.int32)
    x1c, y1c, x2c, y2c = (a[:, None] for a in (x1, y1, x2, y2))
    x1r, y1r, x2r, y2r, scr = (a[None, :] for a in (x1, y1, x2, y2, sc))
    gir = gip[None, :]
    orr = orp[None, :]
    gx1, gy1, gx2, gy2 = (gtb[:, k][:, None] for k in range(4))

    m = _pack_call(x1c, y1c, x2c, y2c, x1r, y1r, x2r, y2r)
    plv, pb, kp = _aux_call(x1r, y1r, x2r, y2r, scr, gir, orr, gx1, gy1, gx2, gy2)
    out = _scan_call(gir, plv, kp, pb, m)
    return out[0, :]


# speedup vs baseline: 432.7369x; 1.0043x over previous
import functools

import jax
import jax.numpy as jnp
from jax.experimental import pallas as pl
from jax.experimental.pallas import tpu as pltpu
from jax.experimental.pallas import tpu_sc as plsc

NMS_THR = 0.5
MIN_HEIGHT = 50.0
EPS = 1e-6
N = 5000
G = 128
NP = 5120
TB = 512
W16 = NP // 16
WB = TB // 16


def _pack_body(x1c, y1c, x2c, y2c, x1r, y1r, x2r, y2r, m_ref):
    ib = pl.program_id(0)
    x1i = x1c[...]; y1i = y1c[...]; x2i = x2c[...]; y2i = y2c[...]
    a_i = (x2i - x1i) * (y2i - y1i)
    row_iota = jax.lax.broadcasted_iota(jnp.int32, (TB, TB), 0)
    col_iota = jax.lax.broadcasted_iota(jnp.int32, (TB, TB), 1)
    tri = col_iota > row_iota
    ci = jax.lax.broadcasted_iota(jnp.int32, (TB, WB), 0)
    wi = jax.lax.broadcasted_iota(jnp.int32, (TB, WB), 1)
    pmat = jnp.where((ci // 16) == wi,
                     jnp.left_shift(jnp.int32(1), ci % 16), 0).astype(jnp.bfloat16)

    for jb in range(NP // TB):
        sl = slice(jb * WB, (jb + 1) * WB)

        @pl.when(jb >= ib)
        def _(jb=jb, sl=sl):
            cs = slice(jb * TB, (jb + 1) * TB)
            x1j = x1r[:, cs]; y1j = y1r[:, cs]
            x2j = x2r[:, cs]; y2j = y2r[:, cs]
            a_j = (x2j - x1j) * (y2j - y1j)
            ltx = jnp.maximum(x1i, x1j)
            lty = jnp.maximum(y1i, y1j)
            rbx = jnp.minimum(x2i, x2j)
            rby = jnp.minimum(y2i, y2j)
            wx = jnp.clip(rbx - ltx, 0.0, None)
            wy = jnp.clip(rby - lty, 0.0, None)
            inter = wx * wy
            union = a_i + a_j - inter
            iou = inter / jnp.maximum(union, EPS)
            over = iou > NMS_THR
            if jb == NP // TB - 1:
                over = over & (jb * TB + col_iota < N)
            over = over & (tri | (jb > ib))
            bits = jax.lax.dot_general(over.astype(jnp.bfloat16), pmat,
                                       (((1,), (0,)), ((), ())),
                                       preferred_element_type=jnp.float32)
            m_ref[:, sl] = bits.astype(jnp.int32)

        @pl.when(jb < ib)
        def _(sl=sl):
            m_ref[:, sl] = jnp.zeros((TB, WB), jnp.int32)


def _aux_body(x1r, y1r, x2r, y2r, scr, gir, orr, gx1, gy1, gx2, gy2,
              pl_out, pb_out, kp_out):
    gidx = jax.lax.broadcasted_iota(jnp.int32, (G, 1), 0)
    gi = gir[...]
    eq = gi == gidx
    ggx1 = jnp.sum(jnp.where(eq, gx1[...], 0.0), axis=0, keepdims=True)
    ggy1 = jnp.sum(jnp.where(eq, gy1[...], 0.0), axis=0, keepdims=True)
    ggx2 = jnp.sum(jnp.where(eq, gx2[...], 0.0), axis=0, keepdims=True)
    ggy2 = jnp.sum(jnp.where(eq, gy2[...], 0.0), axis=0, keepdims=True)
    bx1 = x1r[...]; by1 = y1r[...]; bx2 = x2r[...]; by2 = y2r[...]
    area_g = (ggx2 - ggx1) * (ggy2 - ggy1)
    area_b = (bx2 - bx1) * (by2 - by1)
    ltx = jnp.maximum(ggx1, bx1)
    lty = jnp.maximum(ggy1, by1)
    rbx = jnp.minimum(ggx2, bx2)
    rby = jnp.minimum(ggy2, by2)
    wx = jnp.clip(rbx - ltx, 0.0, None)
    wy = jnp.clip(rby - lty, 0.0, None)
    inter = wx * wy
    msi = inter / jnp.maximum(area_g + area_b - inter, EPS)
    msi = jnp.clip(msi, EPS, None)
    pl_out[...] = -jnp.log(jnp.clip(1.0 - NMS_THR + msi, None, 1.0)) * scr[...]
    minf = jnp.float32(-jnp.inf)
    msc = jnp.where(eq, scr[...], minf)
    best = jnp.max(msc, axis=1, keepdims=True)
    cand2 = eq & (msc == best)
    oid = orr[...]
    pi = jnp.min(jnp.where(cand2, oid, N), axis=1, keepdims=True)
    sel = cand2 & (oid == pi)
    sx1 = jnp.sum(jnp.where(sel, bx1, 0.0), axis=1, keepdims=True)
    sy1 = jnp.sum(jnp.where(sel, by1, 0.0), axis=1, keepdims=True)
    sx2 = jnp.sum(jnp.where(sel, bx2, 0.0), axis=1, keepdims=True)
    sy2 = jnp.sum(jnp.where(sel, by2, 0.0), axis=1, keepdims=True)
    g_area = (gx2[...] - gx1[...]) * (gy2[...] - gy1[...])
    s_area = (sx2 - sx1) * (sy2 - sy1)
    ltx2 = jnp.maximum(gx1[...], sx1)
    lty2 = jnp.maximum(gy1[...], sy1)
    rbx2 = jnp.minimum(gx2[...], sx2)
    rby2 = jnp.minimum(gy2[...], sy2)
    wx2 = jnp.clip(rbx2 - ltx2, 0.0, None)
    wy2 = jnp.clip(rby2 - lty2, 0.0, None)
    inter2 = wx2 * wy2
    gsel = inter2 / jnp.maximum(g_area + s_area - inter2, EPS)
    has = jnp.any(eq, axis=1, keepdims=True)
    height = gy2[...] - gy1[...]
    kp_out[...] = (has & (height >= MIN_HEIGHT)).astype(jnp.int32)
    pb_out[...] = jnp.where(has, 1.0 - gsel, 0.0)


def _scan_body(gi_sref, pl_sref, kp_sref, pb_sref, m_ref, out_sref,
               acc_ref, accp_ref, seen_scr):
    def init_g(g, c):
        seen_scr[0, g] = 0
        return c
    jax.lax.fori_loop(0, G, init_g, 0)
    lanes = jax.lax.broadcasted_iota(jnp.int32, (1, W16), 1)
    NW = (N + 15) // 16
    base = lanes * 16
    svalid = jnp.clip(N - base, 0, 16)
    acc_ref[...] = jnp.bitwise_not(
        jnp.left_shift(jnp.int32(1), svalid) - 1) & 0xFFFF
    accp_ref[...] = jnp.zeros((1, W16), jnp.int32)

    def _extract(acc, w):
        return jnp.sum(acc * (lanes == w).astype(jnp.int32))

    def wcond(st):
        return st[0] < NW

    def wbody(st):
        w, word, pcnt, psum, l_cnt, l_pl, l_p = st
        free = jnp.bitwise_not(word) & 0xFFFF

        def advance():
            return (w + 1, _extract(acc_ref[...], w + 1),
                    pcnt, psum, l_cnt, l_pl, l_p)

        def pick():
            low = free & (-free)
            b = (jax.lax.bitcast_convert_type(
                low.astype(jnp.float32), jnp.int32) >> 23) - 127
            p = w * 16 + b
            gv = gi_sref[0, p]
            counted = seen_scr[0, gv]
            seen_scr[0, gv] = 1
            acc = acc_ref[...]
            row = m_ref[pl.ds(p, 1), :]
            accp_ref[...] = acc
            acc_new = acc | row
            acc_ref[...] = acc_new
            word2 = _extract(acc_new, w) | word | low
            plv = pl_sref[0, p]
            keptc = (counted > 0).astype(jnp.int32)
            return (w, word2, pcnt + keptc,
                    psum + jnp.where(keptc > 0, plv, 0.0),
                    keptc, plv, p)

        return jax.lax.cond(free == 0, advance, pick)

    _, _, pcnt, psum, l_cnt, l_pl, l_p = jax.lax.while_loop(
        wcond, wbody,
        (jnp.int32(0), _extract(acc_ref[...], 0),
         jnp.int32(0), jnp.float32(0.0),
         jnp.int32(0), jnp.float32(0.0), jnp.int32(0)))
    accp = accp_ref[...]
    base = lanes * 16
    s1 = jnp.clip((l_p + 1) - base, 0, 16)
    mask_lo = jnp.left_shift(jnp.int32(0xFFFF), s1) & 0xFFFF
    s2 = jnp.clip(N - base, 0, 16)
    mask_hi = jnp.left_shift(jnp.int32(1), s2) - 1
    zeros_alive = jnp.bitwise_not(accp) & (mask_lo & mask_hi)
    l_rem = jnp.max(zeros_alive) > 0
    total_pull = psum - jnp.where((l_cnt > 0) & (~l_rem), l_pl, 0.0)
    pull_loss = total_pull / (pcnt.astype(jnp.float32) + EPS)

    def pg(g, c):
        tp, pc = c
        k = (kp_sref[g, 0] > 0) & (seen_scr[0, g] == 0)
        tp = tp + jnp.where(k, pb_sref[g, 0], 0.0)
        pc = pc + jnp.where(k, 1, 0).astype(jnp.int32)
        return tp, pc

    total_push, push_cnt = jax.lax.fori_loop(
        0, G, pg, (jnp.float32(0.0), jnp.int32(0)))
    push_loss = total_push / (push_cnt.astype(jnp.float32) + EPS)
    out_sref[0, 0] = push_loss
    out_sref[0, 1] = pull_loss


_SC_WORKERS = 32
_BPW = NP // _SC_WORKERS


def _sc_gather_call(c0, c1, c2, c3, c4, c5, idx):
    mesh = plsc.VectorSubcoreMesh(core_axis_name="c", subcore_axis_name="s")
    vec = jax.ShapeDtypeStruct((NP,), jnp.float32)

    @functools.partial(
        pl.kernel, mesh=mesh,
        out_type=[vec] * 6,
        scratch_types=[
            pltpu.VMEM((_BPW,), jnp.int32),
            pltpu.VMEM((_BPW,), jnp.float32),
            pltpu.SemaphoreType.DMA,
        ],
    )
    def k(s0, s1, s2, s3, s4, s5, idx_hbm, o0, o1, o2, o3, o4, o5,
          idx_v, col_v, sem):
        wid = jax.lax.axis_index("s") * 2 + jax.lax.axis_index("c")
        base = wid * _BPW
        pltpu.sync_copy(idx_hbm.at[pl.ds(base, _BPW)], idx_v)
        for src, dst in ((s0, o0), (s1, o1), (s2, o2),
                         (s3, o3), (s4, o4), (s5, o5)):
            pltpu.async_copy(src.at[idx_v], col_v, sem).wait()
            pltpu.sync_copy(col_v, dst.at[pl.ds(base, _BPW)])

    return k(c0, c1, c2, c3, c4, c5, idx)


def _pack_call(x1c, y1c, x2c, y2c, x1r, y1r, x2r, y2r):
    col = pl.BlockSpec((TB, 1), lambda i: (i, 0))
    row = pl.BlockSpec((1, NP), lambda i: (0, 0))
    return pl.pallas_call(
        _pack_body,
        grid=(NP // TB,),
        in_specs=[col, col, col, col, row, row, row, row],
        out_specs=pl.BlockSpec((TB, W16), lambda i: (i, 0)),
        out_shape=jax.ShapeDtypeStruct((NP, W16), jnp.int32),
    )(x1c, y1c, x2c, y2c, x1r, y1r, x2r, y2r)


def _aux_call(x1r, y1r, x2r, y2r, scr, gir, orr, gx1, gy1, gx2, gy2):
    return pl.pallas_call(
        _aux_body,
        in_specs=[pl.BlockSpec()] * 11,
        out_specs=[pl.BlockSpec(), pl.BlockSpec(), pl.BlockSpec()],
        out_shape=[
            jax.ShapeDtypeStruct((1, NP), jnp.float32),
            jax.ShapeDtypeStruct((G, 1), jnp.float32),
            jax.ShapeDtypeStruct((G, 1), jnp.int32),
        ],
    )(x1r, y1r, x2r, y2r, scr, gir, orr, gx1, gy1, gx2, gy2)


def _scan_call(gir, plv, kp, pb, m):
    smem = pl.BlockSpec(memory_space=pltpu.SMEM)
    return pl.pallas_call(
        _scan_body,
        in_specs=[smem, smem, smem, smem, pl.BlockSpec()],
        out_specs=smem,
        out_shape=jax.ShapeDtypeStruct((1, 2), jnp.float32),
        scratch_shapes=[
            pltpu.VMEM((1, W16), jnp.int32),
            pltpu.VMEM((1, W16), jnp.int32),
            pltpu.SMEM((1, G), jnp.int32),
        ],
    )(gir, plv, kp, pb, m)


def kernel(gt_inds, anchor_gt_inds, gt_bboxes, proposal_list):
    del gt_inds
    gi0 = anchor_gt_inds[0].astype(jnp.int32)
    props = proposal_list[0]
    gtb = gt_bboxes[0]
    scores = props[:, 4]
    order = jnp.argsort(scores)[::-1].astype(jnp.int32)
    pad = NP - N
    x1u = jnp.pad(props[:, 0], (0, pad))
    y1u = jnp.pad(props[:, 1], (0, pad))
    x2u = jnp.pad(props[:, 2], (0, pad))
    y2u = jnp.pad(props[:, 3], (0, pad))
    scu = jnp.pad(props[:, 4], (0, pad))
    giu = jnp.pad(gi0.astype(jnp.float32), (0, pad), constant_values=-1.0)
    orp = jnp.pad(order, (0, pad), constant_values=N)
    x1, y1, x2, y2, sc, gif = _sc_gather_call(x1u, y1u, x2u, y2u, scu, giu, orp)
    gip = gif.astype(jnp.int32)
    x1c, y1c, x2c, y2c = (a[:, None] for a in (x1, y1, x2, y2))
    x1r, y1r, x2r, y2r, scr = (a[None, :] for a in (x1, y1, x2, y2, sc))
    gir = gip[None, :]
    orr = orp[None, :]
    gx1, gy1, gx2, gy2 = (gtb[:, k][:, None] for k in range(4))

    m = _pack_call(x1c, y1c, x2c, y2c, x1r, y1r, x2r, y2r)
    plv, pb, kp = _aux_call(x1r, y1r, x2r, y2r, scr, gir, orr, gx1, gy1, gx2, gy2)
    out = _scan_call(gir, plv, kp, pb, m)
    return out[0, :]
